# baseline (speedup 1.0000x reference)
import functools

import jax
import jax.numpy as jnp
from jax import lax
from jax.experimental import pallas as pl
from jax.experimental.pallas import tpu as pltpu
from jax.experimental.pallas import tpu_sc as plsc

N_FACTORS = 64
PAIRS = N_FACTORS // 2
LINE = 128
BATCH = 16384
NUM_WORKERS = 32
B_PER_W = BATCH // NUM_WORKERS
CHUNK = 128
N_CHUNKS = B_PER_W // CHUNK
L = 16
HI_MASK = -65536
RB = 4096


def _pack_body(u_ref, i_ref, out_ref):
    eye = jnp.eye(N_FACTORS, dtype=jnp.bfloat16)

    def packT(ref):
        x16 = ref[...].astype(jnp.bfloat16)
        xt = lax.dot_general(x16, eye, (((0,), (0,)), ((), ())),
                             preferred_element_type=jnp.float32)
        return pltpu.bitcast(xt.astype(jnp.bfloat16), jnp.int32)

    out_ref[...] = jnp.concatenate([packT(u_ref), packT(i_ref)], axis=1)


def _bpr_body(users_ref, pos_ref, neg_ref, ctab_ref, out_ref,
              idx_u, idx_p, idx_n, sidx_u, sidx_p, sidx_n,
              rows_u, rows_p, rows_n, out_buf, sem):
    wid = lax.axis_index("s") * 2 + lax.axis_index("c")
    row0 = wid * N_CHUNKS
    base = wid * B_PER_W

    pltpu.sync_copy(users_ref.at[pl.ds(row0, N_CHUNKS)], idx_u)
    pltpu.sync_copy(pos_ref.at[pl.ds(row0, N_CHUNKS)], idx_p)
    pltpu.sync_copy(neg_ref.at[pl.ds(row0, N_CHUNKS)], idx_n)

    for srcr, dst in ((idx_u, sidx_u), (idx_p, sidx_p), (idx_n, sidx_n)):
        for j in range(N_CHUNKS):
            for k in range(CHUNK // L):
                s = pl.ds(k * L, L)
                dst[j, s] = lax.shift_right_logical(srcr[j, s], 1)

    def fire(c):
        buf = pl.ds((c % 2) * CHUNK, CHUNK)
        return (
            pltpu.async_copy(ctab_ref.at[sidx_u.at[c]], rows_u.at[buf], sem),
            pltpu.async_copy(ctab_ref.at[sidx_p.at[c]], rows_p.at[buf], sem),
            pltpu.async_copy(ctab_ref.at[sidx_n.at[c]], rows_n.at[buf], sem),
        )

    def lo(x):
        return plsc.bitcast(lax.shift_left(x, 16), jnp.float32)

    def hi(x):
        return plsc.bitcast(x & HI_MASK, jnp.float32)

    lanes = lax.iota(jnp.int32, L)
    handles = fire(0)
    for c in range(N_CHUNKS):
        next_handles = fire(c + 1) if c + 1 < N_CHUNKS else None
        for h in handles:
            h.wait()
        handles = next_handles

        buf_row0 = (c % 2) * CHUNK

        def group(g, _):
            gs = pl.ds(g * L, L)
            row_vec = buf_row0 + g * L + lanes
            sh_u = (1 - (idx_u[c, gs] & 1)) * 16
            sh_p = (1 - (idx_p[c, gs] & 1)) * 16
            sh_n = (1 - (idx_n[c, gs] & 1)) * 16

            def wstep(w, acc):
                col = (w + lanes) & (N_FACTORS - 1)
                uw = plsc.load_gather(rows_u, [row_vec, col])
                pw = plsc.load_gather(rows_p, [row_vec, col + N_FACTORS])
                nw = plsc.load_gather(rows_n, [row_vec, col + N_FACTORS])
                uu = plsc.bitcast(lax.shift_left(uw, sh_u) & HI_MASK,
                                  jnp.float32)
                pp = plsc.bitcast(lax.shift_left(pw, sh_p) & HI_MASK,
                                  jnp.float32)
                nn = plsc.bitcast(lax.shift_left(nw, sh_n) & HI_MASK,
                                  jnp.float32)
                return acc + uu * (pp - nn)

            acc = lax.fori_loop(0, N_FACTORS, wstep,
                                jnp.zeros((L,), jnp.float32), unroll=8)
            out_buf[pl.ds(c * CHUNK + g * L, L)] = acc
            return _

        lax.fori_loop(0, CHUNK // L, group, None)

    pltpu.sync_copy(out_buf, out_ref.at[pl.ds(base, B_PER_W)])


def kernel(users, pos_items, neg_items, user_table, item_table):
    users = users.astype(jnp.int32).reshape(BATCH // CHUNK, CHUNK)
    pos_items = pos_items.astype(jnp.int32).reshape(BATCH // CHUNK, CHUNK)
    neg_items = neg_items.astype(jnp.int32).reshape(BATCH // CHUNK, CHUNK)
    n_rows = user_table.shape[0]

    pack = pl.pallas_call(
        _pack_body,
        grid=(pl.cdiv(n_rows, RB),),
        in_specs=[pl.BlockSpec((N_FACTORS, RB), lambda j: (0, j)),
                  pl.BlockSpec((N_FACTORS, RB), lambda j: (0, j))],
        out_specs=pl.BlockSpec((RB // 2, LINE), lambda j: (j, 0)),
        out_shape=jax.ShapeDtypeStruct((n_rows // 2, LINE), jnp.int32),
    )
    ctab = pack(user_table.T, item_table.T)

    mesh = plsc.VectorSubcoreMesh(core_axis_name="c", subcore_axis_name="s")
    run = functools.partial(
        pl.kernel,
        mesh=mesh,
        compiler_params=pltpu.CompilerParams(needs_layout_passes=False),
        out_type=jax.ShapeDtypeStruct((BATCH,), jnp.float32),
        scratch_types=[
            pltpu.VMEM((N_CHUNKS, CHUNK), jnp.int32),
            pltpu.VMEM((N_CHUNKS, CHUNK), jnp.int32),
            pltpu.VMEM((N_CHUNKS, CHUNK), jnp.int32),
            pltpu.VMEM((N_CHUNKS, CHUNK), jnp.int32),
            pltpu.VMEM((N_CHUNKS, CHUNK), jnp.int32),
            pltpu.VMEM((N_CHUNKS, CHUNK), jnp.int32),
            pltpu.VMEM((2 * CHUNK, LINE), jnp.int32),
            pltpu.VMEM((2 * CHUNK, LINE), jnp.int32),
            pltpu.VMEM((2 * CHUNK, LINE), jnp.int32),
            pltpu.VMEM((B_PER_W,), jnp.float32),
            pltpu.SemaphoreType.DMA,
        ],
    )(_bpr_body)
    return run(users, pos_items, neg_items, ctab)


# speedup vs baseline: 1.2492x; 1.2492x over previous
import functools

import jax
import jax.numpy as jnp
from jax import lax
from jax.experimental import pallas as pl
from jax.experimental.pallas import tpu as pltpu
from jax.experimental.pallas import tpu_sc as plsc

N_FACTORS = 64
PAIRS = N_FACTORS // 2
LINE = 128
BATCH = 16384
NUM_WORKERS = 32
B_PER_W = BATCH // NUM_WORKERS
CHUNK = 128
N_CHUNKS = B_PER_W // CHUNK
L = 16
HI_MASK = -65536
RB = 8192


def _pack_body(u_ref, i_ref, out_ref, wt_ref):
    u16 = u_ref[...].astype(jnp.bfloat16)
    i16 = i_ref[...].astype(jnp.bfloat16)
    w = jnp.concatenate([pltpu.bitcast(u16, jnp.int32),
                         pltpu.bitcast(i16, jnp.int32)], axis=0)
    wt_ref[...] = w.T
    even = wt_ref[pl.Slice(0, RB // 2, 2), :]
    odd = wt_ref[pl.Slice(1, RB // 2, 2), :]
    out_ref[...] = jnp.concatenate([even, odd], axis=1)


def _bpr_body(users_ref, pos_ref, neg_ref, ctab_ref, out_ref,
              idx_u, idx_p, idx_n, sidx_u, sidx_p, sidx_n,
              rows_u, rows_p, rows_n, out_buf, sem):
    wid = lax.axis_index("s") * 2 + lax.axis_index("c")
    row0 = wid * N_CHUNKS
    base = wid * B_PER_W

    pltpu.sync_copy(users_ref.at[pl.ds(row0, N_CHUNKS)], idx_u)
    pltpu.sync_copy(pos_ref.at[pl.ds(row0, N_CHUNKS)], idx_p)
    pltpu.sync_copy(neg_ref.at[pl.ds(row0, N_CHUNKS)], idx_n)

    for srcr, dst in ((idx_u, sidx_u), (idx_p, sidx_p), (idx_n, sidx_n)):
        for j in range(N_CHUNKS):
            for k in range(CHUNK // L):
                s = pl.ds(k * L, L)
                dst[j, s] = lax.shift_right_logical(srcr[j, s], 1)

    def fire(c):
        buf = pl.ds((c % 2) * CHUNK, CHUNK)
        return (
            pltpu.async_copy(ctab_ref.at[sidx_u.at[c]], rows_u.at[buf], sem),
            pltpu.async_copy(ctab_ref.at[sidx_p.at[c]], rows_p.at[buf], sem),
            pltpu.async_copy(ctab_ref.at[sidx_n.at[c]], rows_n.at[buf], sem),
        )

    def lo(x):
        return plsc.bitcast(lax.shift_left(x, 16), jnp.float32)

    def hi(x):
        return plsc.bitcast(x & HI_MASK, jnp.float32)

    lanes = lax.iota(jnp.int32, L)
    handles = fire(0)
    for c in range(N_CHUNKS):
        next_handles = fire(c + 1) if c + 1 < N_CHUNKS else None
        for h in handles:
            h.wait()
        handles = next_handles

        buf_row0 = (c % 2) * CHUNK

        def group(g, _):
            gs = pl.ds(g * L, L)
            row_vec = buf_row0 + g * L + lanes
            off_u = (idx_u[c, gs] & 1) * (LINE // 2)
            off_p = (idx_p[c, gs] & 1) * (LINE // 2) + PAIRS
            off_n = (idx_n[c, gs] & 1) * (LINE // 2) + PAIRS

            def wstep(w, acc):
                col = (w + lanes) & (PAIRS - 1)
                uw = plsc.load_gather(rows_u, [row_vec, off_u + col])
                pw = plsc.load_gather(rows_p, [row_vec, off_p + col])
                nw = plsc.load_gather(rows_n, [row_vec, off_n + col])
                acc = acc + lo(uw) * (lo(pw) - lo(nw))
                return acc + hi(uw) * (hi(pw) - hi(nw))

            acc = lax.fori_loop(0, PAIRS, wstep,
                                jnp.zeros((L,), jnp.float32), unroll=8)
            out_buf[pl.ds(c * CHUNK + g * L, L)] = acc
            return _

        lax.fori_loop(0, CHUNK // L, group, None)

    pltpu.sync_copy(out_buf, out_ref.at[pl.ds(base, B_PER_W)])


def kernel(users, pos_items, neg_items, user_table, item_table):
    users = users.astype(jnp.int32).reshape(BATCH // CHUNK, CHUNK)
    pos_items = pos_items.astype(jnp.int32).reshape(BATCH // CHUNK, CHUNK)
    neg_items = neg_items.astype(jnp.int32).reshape(BATCH // CHUNK, CHUNK)
    n_rows = user_table.shape[0]

    pack = pl.pallas_call(
        _pack_body,
        grid=(pl.cdiv(n_rows, RB),),
        in_specs=[pl.BlockSpec((N_FACTORS, RB), lambda j: (0, j)),
                  pl.BlockSpec((N_FACTORS, RB), lambda j: (0, j))],
        out_specs=pl.BlockSpec((RB // 2, LINE), lambda j: (j, 0)),
        out_shape=jax.ShapeDtypeStruct((n_rows // 2, LINE), jnp.int32),
        scratch_shapes=[pltpu.VMEM((RB, N_FACTORS), jnp.int32)],
    )
    ctab = pack(user_table.T, item_table.T)

    mesh = plsc.VectorSubcoreMesh(core_axis_name="c", subcore_axis_name="s")
    run = functools.partial(
        pl.kernel,
        mesh=mesh,
        compiler_params=pltpu.CompilerParams(needs_layout_passes=False),
        out_type=jax.ShapeDtypeStruct((BATCH,), jnp.float32),
        scratch_types=[
            pltpu.VMEM((N_CHUNKS, CHUNK), jnp.int32),
            pltpu.VMEM((N_CHUNKS, CHUNK), jnp.int32),
            pltpu.VMEM((N_CHUNKS, CHUNK), jnp.int32),
            pltpu.VMEM((N_CHUNKS, CHUNK), jnp.int32),
            pltpu.VMEM((N_CHUNKS, CHUNK), jnp.int32),
            pltpu.VMEM((N_CHUNKS, CHUNK), jnp.int32),
            pltpu.VMEM((2 * CHUNK, LINE), jnp.int32),
            pltpu.VMEM((2 * CHUNK, LINE), jnp.int32),
            pltpu.VMEM((2 * CHUNK, LINE), jnp.int32),
            pltpu.VMEM((B_PER_W,), jnp.float32),
            pltpu.SemaphoreType.DMA,
        ],
    )(_bpr_body)
    return run(users, pos_items, neg_items, ctab)


# speedup vs baseline: 1.4021x; 1.1224x over previous
import functools

import jax
import jax.numpy as jnp
from jax import lax
from jax.experimental import pallas as pl
from jax.experimental.pallas import tpu as pltpu
from jax.experimental.pallas import tpu_sc as plsc

N_FACTORS = 64
PAIRS = N_FACTORS // 2
LINE = 128
BATCH = 16384
NUM_WORKERS = 32
B_PER_W = BATCH // NUM_WORKERS
CHUNK = 128
N_CHUNKS = B_PER_W // CHUNK
L = 16
HI_MASK = -65536
RB = 16384


def _pack_body(u_ref, i_ref, out_ref, wt_ref):
    u16 = u_ref[...].astype(jnp.bfloat16)
    i16 = i_ref[...].astype(jnp.bfloat16)
    w = jnp.concatenate([pltpu.bitcast(u16, jnp.int32),
                         pltpu.bitcast(i16, jnp.int32)], axis=0)
    wt_ref[...] = w.T
    even = wt_ref[pl.Slice(0, RB // 2, 2), :]
    odd = wt_ref[pl.Slice(1, RB // 2, 2), :]
    out_ref[...] = jnp.concatenate([even, odd], axis=1)


def _bpr_body(users_ref, pos_ref, neg_ref, ctab_ref, out_ref,
              idx_u, idx_p, idx_n, sidx_u, sidx_p, sidx_n,
              rows_u, rows_p, rows_n, out_buf, sem):
    wid = lax.axis_index("s") * 2 + lax.axis_index("c")
    row0 = wid * N_CHUNKS
    base = wid * B_PER_W

    pltpu.sync_copy(users_ref.at[pl.ds(row0, N_CHUNKS)], idx_u)
    pltpu.sync_copy(pos_ref.at[pl.ds(row0, N_CHUNKS)], idx_p)
    pltpu.sync_copy(neg_ref.at[pl.ds(row0, N_CHUNKS)], idx_n)

    for srcr, dst in ((idx_u, sidx_u), (idx_p, sidx_p), (idx_n, sidx_n)):
        for j in range(N_CHUNKS):
            for k in range(CHUNK // L):
                s = pl.ds(k * L, L)
                dst[j, s] = lax.shift_right_logical(srcr[j, s], 1)

    def fire(c):
        buf = pl.ds((c % 2) * CHUNK, CHUNK)
        return (
            pltpu.async_copy(ctab_ref.at[sidx_u.at[c]], rows_u.at[buf], sem),
            pltpu.async_copy(ctab_ref.at[sidx_p.at[c]], rows_p.at[buf], sem),
            pltpu.async_copy(ctab_ref.at[sidx_n.at[c]], rows_n.at[buf], sem),
        )

    def lo(x):
        return plsc.bitcast(lax.shift_left(x, 16), jnp.float32)

    def hi(x):
        return plsc.bitcast(x & HI_MASK, jnp.float32)

    lanes = lax.iota(jnp.int32, L)
    handles = fire(0)
    for c in range(N_CHUNKS):
        next_handles = fire(c + 1) if c + 1 < N_CHUNKS else None
        for h in handles:
            h.wait()
        handles = next_handles

        buf_row0 = (c % 2) * CHUNK

        def group(g, _):
            gs = pl.ds(g * L, L)
            row_vec = buf_row0 + g * L + lanes
            off_u = (idx_u[c, gs] & 1) * (LINE // 2)
            off_p = (idx_p[c, gs] & 1) * (LINE // 2) + PAIRS
            off_n = (idx_n[c, gs] & 1) * (LINE // 2) + PAIRS

            def wstep(w, acc):
                col = (w + lanes) & (PAIRS - 1)
                uw = plsc.load_gather(rows_u, [row_vec, off_u + col])
                pw = plsc.load_gather(rows_p, [row_vec, off_p + col])
                nw = plsc.load_gather(rows_n, [row_vec, off_n + col])
                acc = acc + lo(uw) * (lo(pw) - lo(nw))
                return acc + hi(uw) * (hi(pw) - hi(nw))

            acc = lax.fori_loop(0, PAIRS, wstep,
                                jnp.zeros((L,), jnp.float32), unroll=8)
            out_buf[pl.ds(c * CHUNK + g * L, L)] = acc
            return _

        lax.fori_loop(0, CHUNK // L, group, None)

    pltpu.sync_copy(out_buf, out_ref.at[pl.ds(base, B_PER_W)])


def kernel(users, pos_items, neg_items, user_table, item_table):
    users = users.astype(jnp.int32).reshape(BATCH // CHUNK, CHUNK)
    pos_items = pos_items.astype(jnp.int32).reshape(BATCH // CHUNK, CHUNK)
    neg_items = neg_items.astype(jnp.int32).reshape(BATCH // CHUNK, CHUNK)
    n_rows = user_table.shape[0]

    pack = pl.pallas_call(
        _pack_body,
        grid=(pl.cdiv(n_rows, RB),),
        in_specs=[pl.BlockSpec((N_FACTORS, RB), lambda j: (0, j)),
                  pl.BlockSpec((N_FACTORS, RB), lambda j: (0, j))],
        out_specs=pl.BlockSpec((RB // 2, LINE), lambda j: (j, 0)),
        out_shape=jax.ShapeDtypeStruct((n_rows // 2, LINE), jnp.int32),
        scratch_shapes=[pltpu.VMEM((RB, N_FACTORS), jnp.int32)],
    )
    ctab = pack(user_table.T, item_table.T)

    mesh = plsc.VectorSubcoreMesh(core_axis_name="c", subcore_axis_name="s")
    run = functools.partial(
        pl.kernel,
        mesh=mesh,
        compiler_params=pltpu.CompilerParams(needs_layout_passes=False),
        out_type=jax.ShapeDtypeStruct((BATCH,), jnp.float32),
        scratch_types=[
            pltpu.VMEM((N_CHUNKS, CHUNK), jnp.int32),
            pltpu.VMEM((N_CHUNKS, CHUNK), jnp.int32),
            pltpu.VMEM((N_CHUNKS, CHUNK), jnp.int32),
            pltpu.VMEM((N_CHUNKS, CHUNK), jnp.int32),
            pltpu.VMEM((N_CHUNKS, CHUNK), jnp.int32),
            pltpu.VMEM((N_CHUNKS, CHUNK), jnp.int32),
            pltpu.VMEM((2 * CHUNK, LINE), jnp.int32),
            pltpu.VMEM((2 * CHUNK, LINE), jnp.int32),
            pltpu.VMEM((2 * CHUNK, LINE), jnp.int32),
            pltpu.VMEM((B_PER_W,), jnp.float32),
            pltpu.SemaphoreType.DMA,
        ],
    )(_bpr_body)
    return run(users, pos_items, neg_items, ctab)


# speedup vs baseline: 1.4457x; 1.0311x over previous
import functools

import jax
import jax.numpy as jnp
from jax import lax
from jax.experimental import pallas as pl
from jax.experimental.pallas import tpu as pltpu
from jax.experimental.pallas import tpu_sc as plsc

N_FACTORS = 64
PAIRS = N_FACTORS // 2
LINE = 128
BATCH = 16384
NUM_WORKERS = 32
B_PER_W = BATCH // NUM_WORKERS
CHUNK = 128
N_CHUNKS = B_PER_W // CHUNK
L = 16
HI_MASK = -65536
RB = 24576


def _pack_body(u_ref, i_ref, out_ref, wt_ref):
    u16 = u_ref[...].astype(jnp.bfloat16)
    i16 = i_ref[...].astype(jnp.bfloat16)
    w = jnp.concatenate([pltpu.bitcast(u16, jnp.int32),
                         pltpu.bitcast(i16, jnp.int32)], axis=0)
    wt_ref[...] = w.T
    even = wt_ref[pl.Slice(0, RB // 2, 2), :]
    odd = wt_ref[pl.Slice(1, RB // 2, 2), :]
    out_ref[...] = jnp.concatenate([even, odd], axis=1)


def _bpr_body(users_ref, pos_ref, neg_ref, ctab_ref, out_ref,
              idx_u, idx_p, idx_n, sidx_u, sidx_p, sidx_n,
              rows_u, rows_p, rows_n, out_buf, sem):
    wid = lax.axis_index("s") * 2 + lax.axis_index("c")
    row0 = wid * N_CHUNKS
    base = wid * B_PER_W

    pltpu.sync_copy(users_ref.at[pl.ds(row0, N_CHUNKS)], idx_u)
    pltpu.sync_copy(pos_ref.at[pl.ds(row0, N_CHUNKS)], idx_p)
    pltpu.sync_copy(neg_ref.at[pl.ds(row0, N_CHUNKS)], idx_n)

    for srcr, dst in ((idx_u, sidx_u), (idx_p, sidx_p), (idx_n, sidx_n)):
        for j in range(N_CHUNKS):
            for k in range(CHUNK // L):
                s = pl.ds(k * L, L)
                dst[j, s] = lax.shift_right_logical(srcr[j, s], 1)

    def fire(c):
        buf = pl.ds((c % 2) * CHUNK, CHUNK)
        return (
            pltpu.async_copy(ctab_ref.at[sidx_u.at[c]], rows_u.at[buf], sem),
            pltpu.async_copy(ctab_ref.at[sidx_p.at[c]], rows_p.at[buf], sem),
            pltpu.async_copy(ctab_ref.at[sidx_n.at[c]], rows_n.at[buf], sem),
        )

    def lo(x):
        return plsc.bitcast(lax.shift_left(x, 16), jnp.float32)

    def hi(x):
        return plsc.bitcast(x & HI_MASK, jnp.float32)

    lanes = lax.iota(jnp.int32, L)
    handles = fire(0)
    for c in range(N_CHUNKS):
        next_handles = fire(c + 1) if c + 1 < N_CHUNKS else None
        for h in handles:
            h.wait()
        handles = next_handles

        buf_row0 = (c % 2) * CHUNK

        def group(g, _):
            gs = pl.ds(g * L, L)
            row_vec = buf_row0 + g * L + lanes
            off_u = (idx_u[c, gs] & 1) * (LINE // 2)
            off_p = (idx_p[c, gs] & 1) * (LINE // 2) + PAIRS
            off_n = (idx_n[c, gs] & 1) * (LINE // 2) + PAIRS

            def wstep(w, acc):
                col = (w + lanes) & (PAIRS - 1)
                uw = plsc.load_gather(rows_u, [row_vec, off_u + col])
                pw = plsc.load_gather(rows_p, [row_vec, off_p + col])
                nw = plsc.load_gather(rows_n, [row_vec, off_n + col])
                acc = acc + lo(uw) * (lo(pw) - lo(nw))
                return acc + hi(uw) * (hi(pw) - hi(nw))

            acc = lax.fori_loop(0, PAIRS, wstep,
                                jnp.zeros((L,), jnp.float32), unroll=8)
            out_buf[pl.ds(c * CHUNK + g * L, L)] = acc
            return _

        lax.fori_loop(0, CHUNK // L, group, None)

    pltpu.sync_copy(out_buf, out_ref.at[pl.ds(base, B_PER_W)])


def kernel(users, pos_items, neg_items, user_table, item_table):
    users = users.astype(jnp.int32).reshape(BATCH // CHUNK, CHUNK)
    pos_items = pos_items.astype(jnp.int32).reshape(BATCH // CHUNK, CHUNK)
    neg_items = neg_items.astype(jnp.int32).reshape(BATCH // CHUNK, CHUNK)
    n_rows = user_table.shape[0]

    pack = pl.pallas_call(
        _pack_body,
        grid=(pl.cdiv(n_rows, RB),),
        in_specs=[pl.BlockSpec((N_FACTORS, RB), lambda j: (0, j)),
                  pl.BlockSpec((N_FACTORS, RB), lambda j: (0, j))],
        out_specs=pl.BlockSpec((RB // 2, LINE), lambda j: (j, 0)),
        out_shape=jax.ShapeDtypeStruct((n_rows // 2, LINE), jnp.int32),
        scratch_shapes=[pltpu.VMEM((RB, N_FACTORS), jnp.int32)],
    )
    ctab = pack(user_table.T, item_table.T)

    mesh = plsc.VectorSubcoreMesh(core_axis_name="c", subcore_axis_name="s")
    run = functools.partial(
        pl.kernel,
        mesh=mesh,
        compiler_params=pltpu.CompilerParams(needs_layout_passes=False),
        out_type=jax.ShapeDtypeStruct((BATCH,), jnp.float32),
        scratch_types=[
            pltpu.VMEM((N_CHUNKS, CHUNK), jnp.int32),
            pltpu.VMEM((N_CHUNKS, CHUNK), jnp.int32),
            pltpu.VMEM((N_CHUNKS, CHUNK), jnp.int32),
            pltpu.VMEM((N_CHUNKS, CHUNK), jnp.int32),
            pltpu.VMEM((N_CHUNKS, CHUNK), jnp.int32),
            pltpu.VMEM((N_CHUNKS, CHUNK), jnp.int32),
            pltpu.VMEM((2 * CHUNK, LINE), jnp.int32),
            pltpu.VMEM((2 * CHUNK, LINE), jnp.int32),
            pltpu.VMEM((2 * CHUNK, LINE), jnp.int32),
            pltpu.VMEM((B_PER_W,), jnp.float32),
            pltpu.SemaphoreType.DMA,
        ],
    )(_bpr_body)
    return run(users, pos_items, neg_items, ctab)
